# baseline (speedup 1.0000x reference)
import functools

import jax
import jax.numpy as jnp
from jax import lax
from jax.experimental import pallas as pl
from jax.experimental.pallas import tpu as pltpu

_B = 8
_HP = 16
_PP = _HP * _HP
_C = 256


def _mask_head_kernel(x_ref, wc_ref, wtc_ref, w5b_ref, bias_ref, out_ref, x3h_ref):
    bb = x_ref.shape[0]
    bm = bb * _PP
    ri = lax.broadcasted_iota(jnp.int32, (bm, 1), 0)
    local = ri & (_PP - 1)
    h = local >> 4
    w = local & (_HP - 1)
    interior = (h >= 1) & (h <= 14) & (w >= 1) & (w <= 14)

    dot = functools.partial(jnp.dot, preferred_element_type=jnp.float32)

    x3h_ref[0:16, :] = jnp.zeros((16, 3 * _C), jnp.bfloat16)
    x3h_ref[16 + bm : 32 + bm, :] = jnp.zeros((16, 3 * _C), jnp.bfloat16)

    x = jnp.swapaxes(x_ref[...].astype(jnp.bfloat16), 1, 2).reshape(bm, _C)
    for l in range(4):
        xm1 = pltpu.roll(x, 1, 0)
        xp1 = pltpu.roll(x, bm - 1, 0)
        x3h_ref[16 : 16 + bm, :] = jnp.concatenate([xm1, x, xp1], axis=1)
        acc = dot(x3h_ref[0:bm, :], wc_ref[l, 0])
        acc = acc + dot(x3h_ref[16 : 16 + bm, :], wc_ref[l, 1])
        acc = acc + dot(x3h_ref[32 : 32 + bm, :], wc_ref[l, 2])
        y = acc + bias_ref[l : l + 1, 0:_C]
        x = jnp.where(interior, jnp.maximum(y, 0.0), 0.0).astype(jnp.bfloat16)

    z = jnp.maximum(dot(x, wtc_ref[...]) + bias_ref[4:5, :], 0.0)
    out = jax.nn.sigmoid(dot(z.astype(jnp.bfloat16), w5b_ref[...]) + bias_ref[5:6, 0:12])
    out_ref[...] = out


def kernel(features, w1, b1, w2, b2, w3, b3, w4, b4, wt, bt, w5, b5):
    n = features.shape[0]
    b = _B
    assert n % b == 0
    bm = b * _PP

    xp1_ = jnp.pad(features, ((0, 0), (0, 0), (0, 0), (1, 1)))
    xgrid = jnp.pad(xp1_.reshape(n, _C, 14 * _HP), ((0, 0), (0, 0), (_HP, _HP)))

    wc = jnp.stack(
        [jnp.transpose(wl, (2, 3, 1, 0)).reshape(3, 3 * _C, _C) for wl in (w1, w2, w3, w4)]
    ).astype(jnp.bfloat16)
    wtc = jnp.transpose(wt, (0, 2, 3, 1)).reshape(_C, 4 * _C).astype(jnp.bfloat16)
    w5m = jnp.transpose(w5[:, :, 0, 0])
    w5b = jnp.kron(jnp.eye(4, dtype=w5m.dtype), w5m).astype(jnp.bfloat16)

    bias = jnp.zeros((8, 4 * _C), dtype=jnp.float32)
    bias = bias.at[0:4, 0:_C].set(jnp.stack([b1, b2, b3, b4]))
    bias = bias.at[4, :].set(jnp.tile(bt, 4))
    bias = bias.at[5, 0:12].set(jnp.tile(b5, 4))

    out = pl.pallas_call(
        _mask_head_kernel,
        grid=(n // b,),
        in_specs=[
            pl.BlockSpec((b, _C, _PP), lambda i: (i, 0, 0)),
            pl.BlockSpec((4, 3, 3 * _C, _C), lambda i: (0, 0, 0, 0)),
            pl.BlockSpec((_C, 4 * _C), lambda i: (0, 0)),
            pl.BlockSpec((4 * _C, 12), lambda i: (0, 0)),
            pl.BlockSpec((8, 4 * _C), lambda i: (0, 0)),
        ],
        out_specs=pl.BlockSpec((bm, 12), lambda i: (i, 0)),
        out_shape=jax.ShapeDtypeStruct((n * _PP, 12), jnp.float32),
        scratch_shapes=[pltpu.VMEM((bm + 32, 3 * _C), jnp.bfloat16)],
        compiler_params=pltpu.CompilerParams(
            dimension_semantics=("parallel",),
        ),
    )(xgrid, wc, wtc, w5b, bias)

    m = out.reshape(n, _HP, _HP, 2, 2, 3)[:, 1:15, 1:15]
    return m.transpose(0, 5, 1, 3, 2, 4).reshape(n, 3, 28, 28)


# speedup vs baseline: 1.1024x; 1.1024x over previous
import functools

import jax
import jax.numpy as jnp
from jax import lax
from jax.experimental import pallas as pl
from jax.experimental.pallas import tpu as pltpu

_B = 16
_CH = 2
_HP = 16
_PP = _HP * _HP
_C = 256


def _conv_layers(x, wc_ref, bias_ref, interior):
    bm = x.shape[0]
    dot = functools.partial(jnp.dot, preferred_element_type=jnp.float32)
    for l in range(4):
        shifted = {-1: pltpu.roll(x, 1, 0), 0: x, 1: pltpu.roll(x, bm - 1, 0)}
        acc = None
        for r in (-1, 0, 1):
            p = None
            for c in (-1, 0, 1):
                t = (r + 1) * 3 + (c + 1)
                term = dot(shifted[c], wc_ref[l, t])
                p = term if p is None else p + term
            if r != 0:
                p = pltpu.roll(p, (-16 * r) % bm, 0)
            acc = p if acc is None else acc + p
        y = acc + bias_ref[l : l + 1, 0:_C]
        x = jnp.where(interior, jnp.maximum(y, 0.0), 0.0).astype(jnp.bfloat16)
    return x


def _mask_head_kernel(x_ref, wc_ref, wtc_ref, w5b_ref, bias_ref, out_ref):
    bm = x_ref.shape[0] // _CH
    ri = lax.broadcasted_iota(jnp.int32, (bm, 1), 0)
    local = ri & (_PP - 1)
    h = local >> 4
    w = local & (_HP - 1)
    interior = (h >= 1) & (h <= 14) & (w >= 1) & (w <= 14)

    dot = functools.partial(jnp.dot, preferred_element_type=jnp.float32)

    for k in range(_CH):
        x = x_ref[k * bm : (k + 1) * bm, :].astype(jnp.bfloat16)
        x = _conv_layers(x, wc_ref, bias_ref, interior)
        z = jnp.maximum(dot(x, wtc_ref[...]) + bias_ref[4:5, :], 0.0)
        out = jax.nn.sigmoid(
            dot(z.astype(jnp.bfloat16), w5b_ref[...]) + bias_ref[5:6, 0:12]
        )
        out_ref[k * bm : (k + 1) * bm, :] = out


def kernel(features, w1, b1, w2, b2, w3, b3, w4, b4, wt, bt, w5, b5):
    n = features.shape[0]
    b = _B
    npad = -n % b
    bm = b * _PP

    xt = jnp.transpose(features, (0, 2, 3, 1))
    xpad = jnp.pad(xt, ((0, npad), (1, 1), (1, 1), (0, 0)))
    ng = n + npad
    xrows = xpad.reshape(ng * _PP, _C)

    wc = jnp.stack(
        [jnp.transpose(wl, (2, 3, 1, 0)).reshape(9, _C, _C) for wl in (w1, w2, w3, w4)]
    ).astype(jnp.bfloat16)
    wtc = jnp.transpose(wt, (0, 2, 3, 1)).reshape(_C, 4 * _C).astype(jnp.bfloat16)
    w5m = jnp.transpose(w5[:, :, 0, 0])
    w5b = jnp.kron(jnp.eye(4, dtype=w5m.dtype), w5m).astype(jnp.bfloat16)

    bias = jnp.zeros((8, 4 * _C), dtype=jnp.float32)
    bias = bias.at[0:4, 0:_C].set(jnp.stack([b1, b2, b3, b4]))
    bias = bias.at[4, :].set(jnp.tile(bt, 4))
    bias = bias.at[5, 0:12].set(jnp.tile(b5, 4))

    out = pl.pallas_call(
        _mask_head_kernel,
        grid=(ng // b,),
        in_specs=[
            pl.BlockSpec((bm, _C), lambda i: (i, 0)),
            pl.BlockSpec((4, 9, _C, _C), lambda i: (0, 0, 0, 0)),
            pl.BlockSpec((_C, 4 * _C), lambda i: (0, 0)),
            pl.BlockSpec((4 * _C, 12), lambda i: (0, 0)),
            pl.BlockSpec((8, 4 * _C), lambda i: (0, 0)),
        ],
        out_specs=pl.BlockSpec((bm, 12), lambda i: (i, 0)),
        out_shape=jax.ShapeDtypeStruct((ng * _PP, 12), jnp.float32),
        compiler_params=pltpu.CompilerParams(
            dimension_semantics=("parallel",),
        ),
    )(xrows, wc, wtc, w5b, bias)

    m = out.reshape(ng, _HP, _HP, 2, 2, 3)[:n, 1:15, 1:15]
    return m.transpose(0, 5, 1, 3, 2, 4).reshape(n, 3, 28, 28)


# speedup vs baseline: 1.4894x; 1.3511x over previous
import functools

import jax
import jax.numpy as jnp
from jax import lax
from jax.experimental import pallas as pl
from jax.experimental.pallas import tpu as pltpu

_B = 8
_HP = 16
_PP = _HP * _HP
_C = 256


def _mask_head_kernel(x_ref, wc_ref, wtc_ref, w5b_ref, bias_ref, out_ref):
    bn = x_ref.shape[1]
    pi = lax.broadcasted_iota(jnp.int32, (1, bn), 1)
    local = pi & (_PP - 1)
    h = local >> 4
    w = local & (_HP - 1)
    interior = (h >= 1) & (h <= 14) & (w >= 1) & (w <= 14)

    dot = functools.partial(jnp.dot, preferred_element_type=jnp.float32)

    x = x_ref[...].astype(jnp.bfloat16)
    for l in range(4):
        shifted = {-1: pltpu.roll(x, 1, 1), 0: x, 1: pltpu.roll(x, bn - 1, 1)}
        acc = None
        for r in (-1, 0, 1):
            p = None
            for c in (-1, 0, 1):
                t = (r + 1) * 3 + (c + 1)
                term = dot(wc_ref[l, t], shifted[c])
                p = term if p is None else p + term
            if r != 0:
                p = pltpu.roll(p, (-16 * r) % bn, 1)
            acc = p if acc is None else acc + p
        y = acc + bias_ref[0:_C, l : l + 1]
        x = jnp.where(interior, jnp.maximum(y, 0.0), 0.0).astype(jnp.bfloat16)

    z = jnp.maximum(dot(wtc_ref[...], x) + bias_ref[:, 4:5], 0.0)
    out = jax.nn.sigmoid(
        dot(w5b_ref[...], z.astype(jnp.bfloat16)) + bias_ref[0:16, 5:6]
    )
    out_ref[...] = out


def kernel(features, w1, b1, w2, b2, w3, b3, w4, b4, wt, bt, w5, b5):
    n = features.shape[0]
    b = _B
    npad = -n % b
    bn = b * _PP

    x4 = features.reshape(n, _C, 14, 14)
    xpad = jnp.pad(x4, ((0, npad), (0, 0), (1, 1), (1, 1)))
    ng = n + npad
    xcols = jnp.transpose(xpad.reshape(ng, _C, _PP), (1, 0, 2)).reshape(_C, ng * _PP)

    wc = jnp.stack(
        [jnp.transpose(wl, (2, 3, 0, 1)).reshape(9, _C, _C) for wl in (w1, w2, w3, w4)]
    ).astype(jnp.bfloat16)
    wtc = jnp.transpose(wt, (2, 3, 1, 0)).reshape(4 * _C, _C).astype(jnp.bfloat16)
    w5m = w5[:, :, 0, 0]
    w5b = jnp.kron(jnp.eye(4, dtype=w5m.dtype), w5m)
    w5b = jnp.pad(w5b, ((0, 4), (0, 0))).astype(jnp.bfloat16)

    bias = jnp.zeros((4 * _C, 8), dtype=jnp.float32)
    bias = bias.at[0:_C, 0:4].set(jnp.stack([b1, b2, b3, b4], axis=1))
    bias = bias.at[:, 4].set(jnp.tile(bt, 4))
    bias = bias.at[0:12, 5].set(jnp.tile(b5, 4))

    out = pl.pallas_call(
        _mask_head_kernel,
        grid=(ng // b,),
        in_specs=[
            pl.BlockSpec((_C, bn), lambda i: (0, i)),
            pl.BlockSpec((4, 9, _C, _C), lambda i: (0, 0, 0, 0)),
            pl.BlockSpec((4 * _C, _C), lambda i: (0, 0)),
            pl.BlockSpec((16, 4 * _C), lambda i: (0, 0)),
            pl.BlockSpec((4 * _C, 8), lambda i: (0, 0)),
        ],
        out_specs=pl.BlockSpec((16, bn), lambda i: (0, i)),
        out_shape=jax.ShapeDtypeStruct((16, ng * _PP), jnp.float32),
        compiler_params=pltpu.CompilerParams(
            dimension_semantics=("parallel",),
        ),
    )(xcols, wc, wtc, w5b, bias)

    m = out[0:12].reshape(2, 2, 3, ng, _HP, _HP)[:, :, :, :n, 1:15, 1:15]
    return m.transpose(3, 2, 4, 0, 5, 1).reshape(n, 3, 28, 28)
